# baseline (speedup 1.0000x reference)
import functools

import jax
import jax.numpy as jnp
import numpy as np
from jax.experimental import pallas as pl
from jax.experimental.pallas import tpu as pltpu

_HAAR = 0.5 * np.array(
    [[1, 1, 1, 1],
     [-1, -1, 1, 1],
     [-1, 1, -1, 1],
     [1, -1, -1, 1]], np.float32)


def _fold_haar(w, c):
    a = np.kron(_HAAR.T, np.eye(c, dtype=np.float32))
    w9 = w.astype(jnp.float32).reshape(9, 4 * c, -1)
    return jnp.einsum("pk,tko->tpo", jnp.asarray(a), w9)


def _tap_w(w, c, p, ww):
    cout = w.shape[-1]
    wf = _fold_haar(w, c).reshape(3, 3, 2, 2, c, cout)
    w6 = wf.transpose(0, 3, 1, 2, 4, 5).reshape(6, 6, c, cout)
    j = np.arange(ww)[:, None]
    u = np.arange(p)[None, :]
    dw = j - 2 * u
    sel = np.clip(dw, 0, 5)
    msk = jnp.asarray(((dw >= 0) & (dw <= 5)).astype(np.float32))
    taps = []
    for r in range(2):
        for e in range(3):
            g = w6[2 * e + r][sel] * msk[:, :, None, None]
            taps.append(g.transpose(0, 2, 1, 3).reshape(ww * c, p * cout))
    return jnp.stack(taps).astype(jnp.bfloat16)


def _prep(y, p):
    n, h, w, c = y.shape
    ww, ng, hh = 2 * p + 4, w // (2 * p), h // 2
    yb = jnp.pad(y.astype(jnp.bfloat16), ((0, 0), (0, 0), (2, 2), (0, 0)))
    win = jnp.stack([yb[:, :, 2 * p * g: 2 * p * g + ww, :]
                     for g in range(ng)], axis=2)
    pad = ((0, 0), (1, 1), (0, 0), (0, 0), (0, 0))
    we = jnp.pad(win[:, 0::2], pad).reshape(n, (hh + 2) * ng, ww * c)
    wo = jnp.pad(win[:, 1::2], pad).reshape(n, (hh + 2) * ng, ww * c)
    return we, wo


def _conv6_body(ng, mt, xe_ref, xo_ref, w_ref, s_ref, b_ref, o_ref):
    mv = o_ref.shape[1]
    for m0 in range(0, mv, mt):
        mb = min(mt, mv - m0)
        acc = None
        for r in range(2):
            xr = xe_ref if r == 0 else xo_ref
            for e in range(3):
                t = e * ng + m0
                part = jnp.dot(xr[0, t:t + mb, :], w_ref[r * 3 + e],
                               preferred_element_type=jnp.float32)
                acc = part if acc is None else acc + part
        y = jnp.maximum(acc * s_ref[...] + b_ref[...], 0.0)
        o_ref[0, m0:m0 + mb, :] = y.astype(o_ref.dtype)


def _conv_stage(xe, xo, w6, s, b, hh, ng, mt):
    n, l, k = xe.shape
    nn = w6.shape[-1]
    mv = hh * ng
    return pl.pallas_call(
        functools.partial(_conv6_body, ng, mt),
        out_shape=jax.ShapeDtypeStruct((n, mv, nn), jnp.bfloat16),
        grid=(n,),
        in_specs=[pl.BlockSpec((1, l, k), lambda i: (i, 0, 0)),
                  pl.BlockSpec((1, l, k), lambda i: (i, 0, 0)),
                  pl.BlockSpec((6, k, nn), lambda i: (0, 0, 0)),
                  pl.BlockSpec((1, nn), lambda i: (0, 0)),
                  pl.BlockSpec((1, nn), lambda i: (0, 0))],
        out_specs=pl.BlockSpec((1, mv, nn), lambda i: (i, 0, 0)),
        compiler_params=pltpu.CompilerParams(
            dimension_semantics=("parallel",)),
    )(xe, xo, w6, s, b)


def _tail_body(xe_ref, xo_ref, w4_ref, s4_ref, b4_ref, w5_ref, s5_ref,
               b5_ref, fw_ref, fb_ref, o_ref, scr_ref):
    ng, mv = 8, 128
    acc = None
    for r in range(2):
        xr = xe_ref if r == 0 else xo_ref
        for e in range(3):
            t = e * ng
            part = jnp.dot(xr[0, t:t + mv, :], w4_ref[r * 3 + e],
                           preferred_element_type=jnp.float32)
            acc = part if acc is None else acc + part
    y4 = jnp.maximum(acc * s4_ref[...] + b4_ref[...], 0.0)
    y4 = y4.astype(jnp.bfloat16)
    y4 = y4.reshape(256, 128)

    scr_ref[...] = jnp.zeros((326, 128), jnp.bfloat16)
    for hrow in range(16):
        dst = 20 + hrow * 18
        scr_ref[dst:dst + 16, :] = y4[hrow * 16:(hrow + 1) * 16, :]

    col = jax.lax.broadcasted_iota(jnp.int32, (288, 1), 0) % 18
    interior = jnp.logical_and(col >= 1, col <= 16)
    acc5 = None
    for tap in range(9):
        t = (tap // 3) * 18 + (tap % 3)
        part = jnp.dot(scr_ref[t:t + 288, :], w5_ref[tap],
                       preferred_element_type=jnp.float32)
        acc5 = part if acc5 is None else acc5 + part
    y5 = jnp.maximum(acc5 * s5_ref[...] + b5_ref[...], 0.0)
    y5 = jnp.where(interior, y5, 0.0)

    pooled = jnp.sum(y5, axis=0, keepdims=True) * (1.0 / 256.0)
    z = jnp.dot(pooled, fw_ref[...], preferred_element_type=jnp.float32)
    z = z + fb_ref[...]
    o_ref[0] = 1.0 / (1.0 + jnp.exp(-z))


def _tail_stage(xe, xo, w4, s4, b4, w5, s5, b5, fw, fb):
    n, l, k = xe.shape
    out = pl.pallas_call(
        _tail_body,
        out_shape=jax.ShapeDtypeStruct((n, 1, 1), jnp.float32),
        grid=(n,),
        in_specs=[pl.BlockSpec((1, l, k), lambda i: (i, 0, 0)),
                  pl.BlockSpec((1, l, k), lambda i: (i, 0, 0)),
                  pl.BlockSpec((6, k, 256), lambda i: (0, 0, 0)),
                  pl.BlockSpec((1, 256), lambda i: (0, 0)),
                  pl.BlockSpec((1, 256), lambda i: (0, 0)),
                  pl.BlockSpec((9, 128, 128), lambda i: (0, 0, 0)),
                  pl.BlockSpec((1, 128), lambda i: (0, 0)),
                  pl.BlockSpec((1, 128), lambda i: (0, 0)),
                  pl.BlockSpec((128, 1), lambda i: (0, 0)),
                  pl.BlockSpec((1, 1), lambda i: (0, 0))],
        out_specs=pl.BlockSpec((1, 1, 1), lambda i: (i, 0, 0)),
        scratch_shapes=[pltpu.VMEM((326, 128), jnp.bfloat16)],
        compiler_params=pltpu.CompilerParams(
            dimension_semantics=("parallel",)),
    )(xe, xo, w4, s4, b4, w5, s5, b5, fw, fb)
    return out.reshape(n, 1)


def kernel(x,
           conv1_w, conv1_scale, conv1_bias,
           conv2_w, conv2_scale, conv2_bias,
           conv3_w, conv3_scale, conv3_bias,
           conv4_w, conv4_scale, conv4_bias,
           conv5_w, conv5_scale, conv5_bias,
           fc_w, fc_b):
    n = x.shape[0]
    w1 = _tap_w(conv1_w, 4, 16, 36)
    w2 = _tap_w(conv2_w, 16, 8, 20)
    w3 = _tap_w(conv3_w, 64, 4, 12)
    w4 = _tap_w(conv4_w, 128, 2, 8)
    w5 = conv5_w.astype(jnp.bfloat16).reshape(9, 128, 128)

    def rt(v, p):
        return jnp.tile(v.astype(jnp.float32).reshape(1, -1), (1, p))

    xe, xo = _prep(x, 16)
    y1 = _conv_stage(xe, xo, w1, rt(conv1_scale, 16), rt(conv1_bias, 16),
                     128, 8, 512)
    xe, xo = _prep(y1.reshape(n, 128, 128, 16), 8)
    y2 = _conv_stage(xe, xo, w2, rt(conv2_scale, 8), rt(conv2_bias, 8),
                     64, 8, 256)
    xe, xo = _prep(y2.reshape(n, 64, 64, 64), 4)
    y3 = _conv_stage(xe, xo, w3, rt(conv3_scale, 4), rt(conv3_bias, 4),
                     32, 8, 256)
    xe, xo = _prep(y3.reshape(n, 32, 32, 128), 2)
    return _tail_stage(xe, xo, w4, rt(conv4_scale, 2), rt(conv4_bias, 2),
                       w5, rt(conv5_scale, 2), rt(conv5_bias, 2),
                       fc_w.astype(jnp.float32).reshape(128, 1),
                       fc_b.astype(jnp.float32).reshape(1, 1))


# speedup vs baseline: 1.2804x; 1.2804x over previous
import functools

import jax
import jax.numpy as jnp
import numpy as np
from jax.experimental import pallas as pl
from jax.experimental.pallas import tpu as pltpu

_HAAR = 0.5 * np.array(
    [[1, 1, 1, 1],
     [-1, -1, 1, 1],
     [-1, 1, -1, 1],
     [1, -1, -1, 1]], np.float32)


def _fold_haar(w, c):
    a = np.kron(_HAAR.T, np.eye(c, dtype=np.float32))
    w9 = w.astype(jnp.float32).reshape(9, 4 * c, -1)
    return jnp.einsum("pk,tko->tpo", jnp.asarray(a), w9)


def _tap_w(w, c, p):
    cout = w.shape[-1]
    wf = _fold_haar(w, c).reshape(3, 3, 2, 2, c, cout)
    w6 = wf.transpose(0, 3, 1, 2, 4, 5).reshape(6, 6, c, cout)
    j = np.arange(2 * p)[:, None]
    u = np.arange(p)[None, :]
    taps = []
    for r in range(2):
        for e in range(3):
            for gd in range(3):
                dw = 2 * p * (gd - 1) + j - 2 * u + 2
                sel = np.clip(dw, 0, 5)
                msk = jnp.asarray(((dw >= 0) & (dw <= 5)).astype(np.float32))
                g = w6[2 * e + r][sel] * msk[:, :, None, None]
                taps.append(
                    g.transpose(0, 2, 1, 3).reshape(2 * p * c, p * cout))
    return jnp.stack(taps).astype(jnp.bfloat16)


def _prep(y, p):
    n, h, w, c = y.shape
    ng, hh, k = w // (2 * p), h // 2, 2 * p * c
    yb = jnp.pad(y.astype(jnp.bfloat16),
                 ((0, 0), (0, 0), (2 * p, 2 * p), (0, 0)))
    yb = yb.reshape(n, h, ng + 2, k)
    pad = ((0, 0), (1, 1), (0, 0), (0, 0))
    out = []
    for r in range(2):
        q = jnp.pad(yb[:, r::2], pad).reshape(n, (hh + 2) * (ng + 2), k)
        out.append(jnp.pad(q, ((0, 0), (1, 1), (0, 0))))
    return out


def _conv18_body(ngp, mt, xe_ref, xo_ref, w_ref, s_ref, b_ref, o_ref):
    mv = o_ref.shape[1]
    for m0 in range(0, mv, mt):
        mb = min(mt, mv - m0)
        acc = None
        for r in range(2):
            xr = xe_ref if r == 0 else xo_ref
            for e in range(3):
                for gd in range(3):
                    t = e * ngp + gd + m0
                    part = jnp.dot(xr[0, t:t + mb, :],
                                   w_ref[(r * 3 + e) * 3 + gd],
                                   preferred_element_type=jnp.float32)
                    acc = part if acc is None else acc + part
        y = jnp.maximum(acc * s_ref[...] + b_ref[...], 0.0)
        o_ref[0, m0:m0 + mb, :] = y.astype(o_ref.dtype)


def _conv_stage(xe, xo, w18, s, b, hh, ngp, mt):
    n, l, k = xe.shape
    nn = w18.shape[-1]
    mv = hh * ngp
    return pl.pallas_call(
        functools.partial(_conv18_body, ngp, mt),
        out_shape=jax.ShapeDtypeStruct((n, mv, nn), jnp.bfloat16),
        grid=(n,),
        in_specs=[pl.BlockSpec((1, l, k), lambda i: (i, 0, 0)),
                  pl.BlockSpec((1, l, k), lambda i: (i, 0, 0)),
                  pl.BlockSpec((18, k, nn), lambda i: (0, 0, 0)),
                  pl.BlockSpec((1, nn), lambda i: (0, 0)),
                  pl.BlockSpec((1, nn), lambda i: (0, 0))],
        out_specs=pl.BlockSpec((1, mv, nn), lambda i: (i, 0, 0)),
        compiler_params=pltpu.CompilerParams(
            dimension_semantics=("parallel",)),
    )(xe, xo, w18, s, b)


def _tail_body(xe_ref, xo_ref, w4_ref, s4_ref, b4_ref, w5_ref, s5_ref,
               b5_ref, fw_ref, fb_ref, o_ref, scr_ref):
    ngp, mv = 10, 160
    acc = None
    for r in range(2):
        xr = xe_ref if r == 0 else xo_ref
        for e in range(3):
            for gd in range(3):
                t = e * ngp + gd
                part = jnp.dot(xr[0, t:t + mv, :],
                               w4_ref[(r * 3 + e) * 3 + gd],
                               preferred_element_type=jnp.float32)
                acc = part if acc is None else acc + part
    y4 = jnp.maximum(acc * s4_ref[...] + b4_ref[...], 0.0)
    y4 = y4.astype(jnp.bfloat16).reshape(16, 10, 2, 128)
    y4 = y4[:, 1:9].reshape(256, 128)

    scr_ref[...] = jnp.zeros((326, 128), jnp.bfloat16)
    for hrow in range(16):
        dst = 20 + hrow * 18
        scr_ref[dst:dst + 16, :] = y4[hrow * 16:(hrow + 1) * 16, :]

    col = jax.lax.broadcasted_iota(jnp.int32, (288, 1), 0) % 18
    interior = jnp.logical_and(col >= 1, col <= 16)
    acc5 = None
    for tap in range(9):
        t = (tap // 3) * 18 + (tap % 3)
        part = jnp.dot(scr_ref[t:t + 288, :], w5_ref[tap],
                       preferred_element_type=jnp.float32)
        acc5 = part if acc5 is None else acc5 + part
    y5 = jnp.maximum(acc5 * s5_ref[...] + b5_ref[...], 0.0)
    y5 = jnp.where(interior, y5, 0.0)

    pooled = jnp.sum(y5, axis=0, keepdims=True) * (1.0 / 256.0)
    z = jnp.dot(pooled, fw_ref[...], preferred_element_type=jnp.float32)
    z = z + fb_ref[...]
    o_ref[0] = 1.0 / (1.0 + jnp.exp(-z))


def _tail_stage(xe, xo, w4, s4, b4, w5, s5, b5, fw, fb):
    n, l, k = xe.shape
    out = pl.pallas_call(
        _tail_body,
        out_shape=jax.ShapeDtypeStruct((n, 1, 1), jnp.float32),
        grid=(n,),
        in_specs=[pl.BlockSpec((1, l, k), lambda i: (i, 0, 0)),
                  pl.BlockSpec((1, l, k), lambda i: (i, 0, 0)),
                  pl.BlockSpec((18, k, 256), lambda i: (0, 0, 0)),
                  pl.BlockSpec((1, 256), lambda i: (0, 0)),
                  pl.BlockSpec((1, 256), lambda i: (0, 0)),
                  pl.BlockSpec((9, 128, 128), lambda i: (0, 0, 0)),
                  pl.BlockSpec((1, 128), lambda i: (0, 0)),
                  pl.BlockSpec((1, 128), lambda i: (0, 0)),
                  pl.BlockSpec((128, 1), lambda i: (0, 0)),
                  pl.BlockSpec((1, 1), lambda i: (0, 0))],
        out_specs=pl.BlockSpec((1, 1, 1), lambda i: (i, 0, 0)),
        scratch_shapes=[pltpu.VMEM((326, 128), jnp.bfloat16)],
        compiler_params=pltpu.CompilerParams(
            dimension_semantics=("parallel",)),
    )(xe, xo, w4, s4, b4, w5, s5, b5, fw, fb)
    return out.reshape(n, 1)


def kernel(x,
           conv1_w, conv1_scale, conv1_bias,
           conv2_w, conv2_scale, conv2_bias,
           conv3_w, conv3_scale, conv3_bias,
           conv4_w, conv4_scale, conv4_bias,
           conv5_w, conv5_scale, conv5_bias,
           fc_w, fc_b):
    n = x.shape[0]
    w1 = _tap_w(conv1_w, 4, 16)
    w2 = _tap_w(conv2_w, 16, 8)
    w3 = _tap_w(conv3_w, 64, 4)
    w4 = _tap_w(conv4_w, 128, 2)
    w5 = conv5_w.astype(jnp.bfloat16).reshape(9, 128, 128)

    def rt(v, p):
        return jnp.tile(v.astype(jnp.float32).reshape(1, -1), (1, p))

    xe, xo = _prep(x, 16)
    y1 = _conv_stage(xe, xo, w1, rt(conv1_scale, 16), rt(conv1_bias, 16),
                     128, 10, 640)
    y1 = y1.reshape(n, 128, 10, 16, 16)[:, :, 1:9].reshape(n, 128, 128, 16)
    xe, xo = _prep(y1, 8)
    y2 = _conv_stage(xe, xo, w2, rt(conv2_scale, 8), rt(conv2_bias, 8),
                     64, 10, 320)
    y2 = y2.reshape(n, 64, 10, 8, 64)[:, :, 1:9].reshape(n, 64, 64, 64)
    xe, xo = _prep(y2, 4)
    y3 = _conv_stage(xe, xo, w3, rt(conv3_scale, 4), rt(conv3_bias, 4),
                     32, 10, 320)
    y3 = y3.reshape(n, 32, 10, 4, 128)[:, :, 1:9].reshape(n, 32, 32, 128)
    xe, xo = _prep(y3, 2)
    return _tail_stage(xe, xo, w4, rt(conv4_scale, 2), rt(conv4_bias, 2),
                       w5, rt(conv5_scale, 1), rt(conv5_bias, 1),
                       fc_w.astype(jnp.float32).reshape(128, 1),
                       fc_b.astype(jnp.float32).reshape(1, 1))


# speedup vs baseline: 3.2591x; 2.5453x over previous
import functools

import jax
import jax.numpy as jnp
import numpy as np
from jax.experimental import pallas as pl
from jax.experimental.pallas import tpu as pltpu

_HAAR = 0.5 * np.array(
    [[1, 1, 1, 1],
     [-1, 1, -1, 1],
     [-1, -1, 1, 1],
     [1, -1, -1, 1]], np.float32)


def _fold_haar(w, c_quarter):
    a = np.kron(_HAAR.T, np.eye(c_quarter, dtype=np.float32))
    w9 = w.astype(jnp.float32).reshape(9, 4 * c_quarter, -1)
    return jnp.einsum("pk,tko->tpo", jnp.asarray(a), w9)


def _pack_w(w9, p):
    _, k, co = w9.shape
    w33 = w9.reshape(3, 3, k, co)
    ui = np.arange(p)[:, None]
    uo = np.arange(p)[None, :]
    taps = []
    for dy in range(3):
        for gd in range(3):
            delta = p * (gd - 1) + ui - uo
            sel = np.clip(delta + 1, 0, 2)
            msk = jnp.asarray((np.abs(delta) <= 1).astype(np.float32))
            g = w33[dy][sel] * msk[:, :, None, None]
            taps.append(g.transpose(0, 2, 1, 3).reshape(p * k, p * co))
    return jnp.stack(taps).astype(jnp.bfloat16)


def _deint_pack_pad(y, hh, wh, p):
    n, _, _, c = y.shape
    yr = y.reshape(n, hh, 2, wh, 2, c)
    q = jnp.transpose(yr, (0, 1, 3, 2, 4, 5))
    q = q.astype(jnp.bfloat16).reshape(n, hh, wh // p, 4 * c * p)
    q = jnp.pad(q, ((0, 0), (1, 1), (1, 1), (0, 0)))
    q = q.reshape(n, (hh + 2) * (wh // p + 2), 4 * c * p)
    return jnp.pad(q, ((0, 0), (1, 1), (0, 0)))


def _unpack(o, h, wgp, p, co):
    n = o.shape[0]
    o = o.reshape(n, h, wgp, p, co)[:, :, 1:wgp - 1]
    return o.reshape(n, h, (wgp - 2) * p, co)


def _conv_body(h, wp, mt, x_ref, w_ref, s_ref, b_ref, o_ref):
    mv = h * wp
    for m0 in range(0, mv, mt):
        mb = min(mt, mv - m0)
        acc = None
        for dy in range(3):
            for dx in range(3):
                t = dy * wp + dx + m0
                part = jnp.dot(x_ref[0, t:t + mb, :], w_ref[dy * 3 + dx],
                               preferred_element_type=jnp.float32)
                acc = part if acc is None else acc + part
        y = jnp.maximum(acc * s_ref[...] + b_ref[...], 0.0)
        o_ref[0, m0:m0 + mb, :] = y.astype(o_ref.dtype)


def _conv_stage(xf, w9, s, b, h, wp, mt):
    n, l, cin = xf.shape
    cout = w9.shape[-1]
    mv = h * wp
    return pl.pallas_call(
        functools.partial(_conv_body, h, wp, mt),
        out_shape=jax.ShapeDtypeStruct((n, mv, cout), jnp.bfloat16),
        grid=(n,),
        in_specs=[pl.BlockSpec((1, l, cin), lambda i: (i, 0, 0)),
                  pl.BlockSpec((9, cin, cout), lambda i: (0, 0, 0)),
                  pl.BlockSpec((1, cout), lambda i: (0, 0)),
                  pl.BlockSpec((1, cout), lambda i: (0, 0))],
        out_specs=pl.BlockSpec((1, mv, cout), lambda i: (i, 0, 0)),
        compiler_params=pltpu.CompilerParams(
            dimension_semantics=("parallel",)),
    )(xf, w9, s, b)


def _tail_body(x_ref, w4_ref, s4_ref, b4_ref, w5_ref, s5_ref, b5_ref,
               fw_ref, fb_ref, o_ref, scr_ref):
    wp, mv = 10, 160
    col = jax.lax.broadcasted_iota(jnp.int32, (mv, 1), 0) % wp
    interior = jnp.logical_and(col >= 1, col <= 8)

    acc = None
    for tap in range(9):
        t = (tap // 3) * wp + (tap % 3)
        part = jnp.dot(x_ref[0, t:t + mv, :], w4_ref[tap],
                       preferred_element_type=jnp.float32)
        acc = part if acc is None else acc + part
    y4 = jnp.maximum(acc * s4_ref[...] + b4_ref[...], 0.0)
    y4 = jnp.where(interior, y4, 0.0).astype(jnp.bfloat16)

    scr_ref[0:11, :] = jnp.zeros((11, 256), jnp.bfloat16)
    scr_ref[171:182, :] = jnp.zeros((11, 256), jnp.bfloat16)
    scr_ref[11:171, :] = y4

    acc5 = None
    for tap in range(9):
        t = (tap // 3) * wp + (tap % 3)
        part = jnp.dot(scr_ref[t:t + mv, :], w5_ref[tap],
                       preferred_element_type=jnp.float32)
        acc5 = part if acc5 is None else acc5 + part
    y5 = jnp.maximum(acc5 * s5_ref[...] + b5_ref[...], 0.0)
    y5 = jnp.where(interior, y5, 0.0)

    pooled = jnp.sum(y5, axis=0, keepdims=True) * (1.0 / 256.0)
    pooled = pooled[:, 0:128] + pooled[:, 128:256]
    z = jnp.dot(pooled, fw_ref[...], preferred_element_type=jnp.float32)
    z = z + fb_ref[...]
    o_ref[0] = 1.0 / (1.0 + jnp.exp(-z))


def _tail_stage(xf, w4, s4, b4, w5, s5, b5, fw, fb):
    n, l, cin = xf.shape
    out = pl.pallas_call(
        _tail_body,
        out_shape=jax.ShapeDtypeStruct((n, 1, 1), jnp.float32),
        grid=(n,),
        in_specs=[pl.BlockSpec((1, l, cin), lambda i: (i, 0, 0)),
                  pl.BlockSpec((9, cin, 256), lambda i: (0, 0, 0)),
                  pl.BlockSpec((1, 256), lambda i: (0, 0)),
                  pl.BlockSpec((1, 256), lambda i: (0, 0)),
                  pl.BlockSpec((9, 256, 256), lambda i: (0, 0, 0)),
                  pl.BlockSpec((1, 256), lambda i: (0, 0)),
                  pl.BlockSpec((1, 256), lambda i: (0, 0)),
                  pl.BlockSpec((128, 1), lambda i: (0, 0)),
                  pl.BlockSpec((1, 1), lambda i: (0, 0))],
        out_specs=pl.BlockSpec((1, 1, 1), lambda i: (i, 0, 0)),
        scratch_shapes=[pltpu.VMEM((182, 256), jnp.bfloat16)],
        compiler_params=pltpu.CompilerParams(
            dimension_semantics=("parallel",)),
    )(xf, w4, s4, b4, w5, s5, b5, fw, fb)
    return out.reshape(n, 1)


def kernel(x,
           conv1_w, conv1_scale, conv1_bias,
           conv2_w, conv2_scale, conv2_bias,
           conv3_w, conv3_scale, conv3_bias,
           conv4_w, conv4_scale, conv4_bias,
           conv5_w, conv5_scale, conv5_bias,
           fc_w, fc_b):
    n = x.shape[0]
    w1 = _pack_w(_fold_haar(conv1_w, 4), 8)
    w2 = _pack_w(_fold_haar(conv2_w, 16), 2)
    w3 = _pack_w(_fold_haar(conv3_w, 64), 2)
    w4 = _pack_w(_fold_haar(conv4_w, 128), 2)
    w5 = _pack_w(conv5_w.astype(jnp.float32).reshape(9, 128, 128), 2)

    def rt(v, p):
        return jnp.tile(v.astype(jnp.float32).reshape(1, -1), (1, p))

    xf = _deint_pack_pad(x, 128, 128, 8)
    y1 = _conv_stage(xf, w1, rt(conv1_scale, 8), rt(conv1_bias, 8),
                     128, 18, 1152)
    xf = _deint_pack_pad(_unpack(y1, 128, 18, 8, 16), 64, 64, 2)
    y2 = _conv_stage(xf, w2, rt(conv2_scale, 2), rt(conv2_bias, 2),
                     64, 34, 1088)
    xf = _deint_pack_pad(_unpack(y2, 64, 34, 2, 64), 32, 32, 2)
    y3 = _conv_stage(xf, w3, rt(conv3_scale, 2), rt(conv3_bias, 2),
                     32, 18, 576)
    xf = _deint_pack_pad(_unpack(y3, 32, 18, 2, 128), 16, 16, 2)
    return _tail_stage(xf, w4, rt(conv4_scale, 2), rt(conv4_bias, 2),
                       w5, rt(conv5_scale, 2), rt(conv5_bias, 2),
                       fc_w.astype(jnp.float32).reshape(128, 1),
                       fc_b.astype(jnp.float32).reshape(1, 1))
